# baseline (speedup 1.0000x reference)
import jax
import jax.numpy as jnp
from jax.experimental import pallas as pl
from jax.experimental.pallas import tpu as pltpu


def _mlp_block_kernel(xt_ref, w1_ref, b1_ref, w2_ref, b2_ref, out_ref):
    in_dim = xt_ref.shape[0]
    nb = xt_ref.shape[1]
    hid = w1_ref.shape[1]

    acc = jnp.broadcast_to(b1_ref[0][None, None, :], (nb, xt_ref.shape[2], hid))
    for d in range(in_dim):
        acc = acc + xt_ref[d][:, :, None] * w1_ref[d, :][None, None, :]
    h = jnp.maximum(acc, 0.0)

    out = jnp.dot(h.reshape(-1, hid), w2_ref[0],
                  preferred_element_type=jnp.float32)
    out_ref[:, 0, :, :] = out.reshape(nb, -1, hid) + b2_ref[0]


def kernel(input, W1, b1, W2, b2):
    B, N, in_dim = input.shape
    hid = W1.shape[1]
    hist = W2.shape[1] // hid

    nb = 1000

    xt = jnp.transpose(input, (2, 1, 0))
    w2r = jnp.transpose(W2.reshape(hid, hist, hid), (1, 0, 2))
    b1r = b1.reshape(1, hid)
    b2r = b2.reshape(hist, 1, hid)

    grid = (N // nb, hist)
    out = pl.pallas_call(
        _mlp_block_kernel,
        grid=grid,
        in_specs=[
            pl.BlockSpec((in_dim, nb, B), lambda i, t: (0, i, 0)),
            pl.BlockSpec((in_dim, hid), lambda i, t: (0, 0)),
            pl.BlockSpec((1, hid), lambda i, t: (0, 0)),
            pl.BlockSpec((1, hid, hid), lambda i, t: (t, 0, 0)),
            pl.BlockSpec((1, 1, hid), lambda i, t: (t, 0, 0)),
        ],
        out_specs=pl.BlockSpec((nb, 1, B, hid), lambda i, t: (i, t, 0, 0)),
        out_shape=jax.ShapeDtypeStruct((N, hist, B, hid), jnp.float32),
        compiler_params=pltpu.CompilerParams(
            dimension_semantics=("parallel", "arbitrary"),
        ),
    )(xt, W1, b1r, w2r, b2r)
    return out


# speedup vs baseline: 3.9208x; 3.9208x over previous
import jax
import jax.numpy as jnp
from jax.experimental import pallas as pl
from jax.experimental.pallas import tpu as pltpu


def _mlp_kernel(xc_ref, e_ref, b1t_ref, g_ref, b2t_ref, out_ref):
    bh = e_ref.shape[1]
    gw = g_ref.shape[0]
    hist_gw = g_ref.shape[1]
    n_groups = bh // gw

    h = jnp.maximum(
        jnp.dot(xc_ref[...], e_ref[...], preferred_element_type=jnp.float32)
        + b1t_ref[0][None, :],
        0.0,
    )

    hist = hist_gw // gw
    for g in range(n_groups):
        og = jnp.dot(h[:, g * gw:(g + 1) * gw], g_ref[...],
                     preferred_element_type=jnp.float32)
        for t in range(hist):
            lo = t * bh + g * gw
            out_ref[:, lo:lo + gw] = (
                og[:, t * gw:(t + 1) * gw] + b2t_ref[0][None, lo:lo + gw]
            )


def kernel(input, W1, b1, W2, b2):
    B, N, in_dim = input.shape
    hid = W1.shape[1]
    hist = W2.shape[1] // hid

    nb = 1000

    xc = jnp.transpose(input, (1, 2, 0)).reshape(N, in_dim * B)
    eye_b = jnp.eye(B, dtype=jnp.float32)
    e_mat = jnp.einsum('bc,dk->dbck', eye_b, W1).reshape(in_dim * B, B * hid)
    b1t = jnp.tile(b1, B).reshape(1, B * hid)
    w2r = W2.reshape(hid, hist, hid)
    eye4 = jnp.eye(4, dtype=jnp.float32)
    g_mat = jnp.einsum('bc,ktj->bktcj', eye4, w2r).reshape(4 * hid,
                                                           hist * 4 * hid)
    b2t = jnp.tile(b2.reshape(hist, 1, hid), (1, B, 1)).reshape(1,
                                                                hist * B * hid)

    out = pl.pallas_call(
        _mlp_kernel,
        grid=(N // nb,),
        in_specs=[
            pl.BlockSpec((nb, in_dim * B), lambda i: (i, 0)),
            pl.BlockSpec((in_dim * B, B * hid), lambda i: (0, 0)),
            pl.BlockSpec((1, B * hid), lambda i: (0, 0)),
            pl.BlockSpec((4 * hid, hist * 4 * hid), lambda i: (0, 0)),
            pl.BlockSpec((1, hist * B * hid), lambda i: (0, 0)),
        ],
        out_specs=pl.BlockSpec((nb, hist * B * hid), lambda i: (i, 0)),
        out_shape=jax.ShapeDtypeStruct((N, hist * B * hid), jnp.float32),
        compiler_params=pltpu.CompilerParams(
            dimension_semantics=("parallel",),
        ),
    )(xc, e_mat, b1t, g_mat, b2t)
    return out.reshape(N, hist, B, hid)
